# baseline (speedup 1.0000x reference)
import functools

import jax
import jax.numpy as jnp
from jax import lax
from jax.experimental import pallas as pl
from jax.experimental.pallas import tpu as pltpu
from jax.experimental.pallas import tpu_sc as plsc

_NUM_CODONS = 64
_EMBED_DIM = 16

_B = 16384
_T = 200
_N = _B * _T

_INFO = plsc.get_sparse_core_info()
_NC = _INFO.num_cores
_NS = _INFO.num_subcores
_NW = _NC * _NS
_BW = _B // _NW
_GRP = 16
_NG = _BW // _GRP


def _gather_kernel(xt_hbm, tableT_hbm, out_hbm, table_v, idx0_v, idx1_v,
                   buf00_v, buf01_v, buf10_v, buf11_v,
                   isem0, isem1, sem00, sem01, sem10, sem11):
    wid = lax.axis_index("s") * _NC + lax.axis_index("c")
    b0 = wid * _BW
    btile0 = wid * (_BW // 128)
    pltpu.sync_copy(tableT_hbm, table_v)
    idxs = (idx0_v, idx1_v)
    isems = (isem0, isem1)
    bufs = ((buf00_v, buf01_v), (buf10_v, buf11_v))
    sems = ((sem00, sem01), (sem10, sem11))

    def idx_copy(t, slot):
        return pltpu.make_async_copy(
            xt_hbm.at[pl.ds(t * _B + b0, _BW)], idxs[slot], isems[slot])

    def compute(idx_v, buf_pair):
        for bg in range(_NG):
            tok = idx_v[pl.ds(bg * _GRP, _GRP)]
            vals = [plsc.load_gather(
                        table_v.at[pl.ds(d * _NUM_CODONS, _NUM_CODONS)], [tok])
                    for d in range(_EMBED_DIM)]
            for d in range(_EMBED_DIM):
                buf_pair[d // 8][bg // 8, d % 8,
                                 pl.ds((bg % 8) * _GRP, _GRP)] = vals[d]

    idx_copy(0, 0).start()

    def body(t, carry):
        par = lax.rem(t, 2)

        @pl.when(t + 1 < _T)
        def _prefetch():
            for slot in range(2):
                @pl.when(par != slot)
                def _go():
                    idx_copy(t + 1, slot).start()

        for slot in range(2):
            @pl.when(par == slot)
            def _step():
                idx_copy(t, slot).wait()
                out_slices = [out_hbm.at[t, db, pl.ds(btile0, _BW // 128)]
                              for db in range(2)]

                @pl.when(t > 1)
                def _drain():
                    for dband in range(2):
                        pltpu.make_async_copy(
                            bufs[slot][dband], out_slices[dband],
                            sems[slot][dband]).wait()

                compute(idxs[slot], bufs[slot])
                for dband in range(2):
                    pltpu.async_copy(bufs[slot][dband], out_slices[dband],
                                     sems[slot][dband])
        return carry

    lax.fori_loop(0, _T, body, 0)
    for slot in range(2):
        for dband in range(2):
            pltpu.make_async_copy(
                bufs[slot][dband],
                out_hbm.at[_T - 2 + slot, dband, pl.ds(btile0, _BW // 128)],
                sems[slot][dband]).wait()


@jax.jit
def _run(xt_flat, tableT_flat):
    mesh = plsc.VectorSubcoreMesh(core_axis_name="c", subcore_axis_name="s")
    kern = functools.partial(
        pl.kernel,
        mesh=mesh,
        out_type=jax.ShapeDtypeStruct((_T, 2, _B // 128, 8, 128), jnp.float32),
        scratch_types=[
            pltpu.VMEM((_NUM_CODONS * _EMBED_DIM,), jnp.float32),
            pltpu.VMEM((_BW,), jnp.int32),
            pltpu.VMEM((_BW,), jnp.int32),
            pltpu.VMEM((_BW // 128, 8, 128), jnp.float32),
            pltpu.VMEM((_BW // 128, 8, 128), jnp.float32),
            pltpu.VMEM((_BW // 128, 8, 128), jnp.float32),
            pltpu.VMEM((_BW // 128, 8, 128), jnp.float32),
            pltpu.SemaphoreType.DMA,
            pltpu.SemaphoreType.DMA,
            pltpu.SemaphoreType.DMA,
            pltpu.SemaphoreType.DMA,
            pltpu.SemaphoreType.DMA,
            pltpu.SemaphoreType.DMA,
        ],
        compiler_params=pltpu.CompilerParams(
            use_tc_tiling_on_sc=True, needs_layout_passes=False),
    )(_gather_kernel)
    return kern(xt_flat, tableT_flat)


def kernel(x, embeddings):
    xt_flat = x.T.reshape(_N)
    tableT_flat = embeddings.T.reshape(_NUM_CODONS * _EMBED_DIM)
    out5 = _run(xt_flat, tableT_flat)
    return out5.transpose(2, 4, 0, 1, 3).reshape(_B, _T, _EMBED_DIM)
